# baseline (speedup 1.0000x reference)
import functools

import jax
import jax.numpy as jnp
from jax import lax
from jax.experimental import pallas as pl
from jax.experimental.pallas import tpu as pltpu
from jax.experimental.pallas import tpu_sc as plsc

_EPS = 1e-3

_N = 10000
_D = 256
_E = 160000
_H = 256
_HH = _H // 2
_NT = 16
_EPT = _E // _NT
_CH = 80
_NCH = _EPT // _CH
_NPAD = 10240
_RPT = _NPAD // _NT
_BLK = 25


def _prepare_body(x_ref, w1_ref, b1_ref, g1_ref, beta1_ref, mm1_ref, mv1_ref,
                  z0_ref, z1_ref):
    s = g1_ref[:] * lax.rsqrt(mv1_ref[:] + _EPS)
    w = w1_ref[:] * s
    b = (b1_ref[:] - mm1_ref[:]) * s + beta1_ref[:]
    y = jnp.dot(x_ref[:], w, preferred_element_type=jnp.float32) + b
    z = jnp.maximum(y, 0.0)
    z0_ref[:] = z[:, :_HH]
    z1_ref[:] = z[:, _HH:]


def _prepare(x, W1, b1, g1, beta1, mm1, mv1):
    return pl.pallas_call(
        _prepare_body,
        out_shape=(
            jax.ShapeDtypeStruct((_N, _HH), jnp.float32),
            jax.ShapeDtypeStruct((_N, _HH), jnp.float32),
        ),
    )(x, W1, b1.reshape(1, _H), g1.reshape(1, _H), beta1.reshape(1, _H),
      mm1.reshape(1, _H), mv1.reshape(1, _H))


def _sc_body(z0_hbm, z1_hbm, src_hbm, dst_hbm, w_hbm,
             sums0_out, sums1_out, cnt_out,
             src_v, dst_v, w_v, rows_v, ones_v, zeros_v, acc, cnt,
             sem_g, sem_s, sem_w, sem_d):
    cid = lax.axis_index("c")
    sid = lax.axis_index("s")

    zeros16 = jnp.zeros((16,), jnp.float32)
    ones16 = jnp.ones((16,), jnp.float32)
    for k in range(_CH // 16):
        ones_v[pl.ds(k * 16, 16)] = ones16
    for k in range(_RPT // 16):
        zeros_v[pl.ds(k * 16, 16)] = zeros16
    for r in range(_CH):
        for g in range(_HH // 16):
            rows_v[r, pl.ds(g * 16, 16)] = zeros16

    for b in range(_RPT // _CH):
        pltpu.sync_copy(rows_v.at[pl.ds(0, _CH)],
                        acc.at[pl.ds(sid * _RPT + b * _CH, _CH)])
    pltpu.sync_copy(zeros_v, cnt.at[pl.ds(sid * _RPT, _RPT)])

    pltpu.sync_copy(src_hbm.at[sid, 0], src_v)
    pltpu.async_copy(w_hbm.at[sid, 0], w_v.at[pl.ds(0, 16)], sem_w)
    pltpu.async_copy(dst_hbm.at[sid, 0], dst_v.at[0], sem_d)

    @pl.when(cid == 0)
    def _():
        pltpu.async_copy(z0_hbm.at[src_v.at[0]], rows_v.at[pl.ds(0, _CH)],
                         sem_g)

    @pl.when(cid == 1)
    def _():
        pltpu.async_copy(z1_hbm.at[src_v.at[0]], rows_v.at[pl.ds(0, _CH)],
                         sem_g)

    plsc.subcore_barrier()

    def _chunk(j, carry):
        b = j % 2
        bn = (j + 1) % 2
        jn = j + 1
        rb = pl.ds(pl.multiple_of(b * _CH, 8), _CH)
        rbn = pl.ds(pl.multiple_of(bn * _CH, 8), _CH)
        wb = pl.ds(pl.multiple_of(b * 16, 8), 16)
        wbn = pl.ds(pl.multiple_of(bn * 16, 8), 16)

        @pl.when(cid == 0)
        def _():
            pltpu.make_async_copy(
                z0_hbm.at[src_v.at[j % _BLK]], rows_v.at[rb], sem_g).wait()

        @pl.when(cid == 1)
        def _():
            pltpu.make_async_copy(
                z1_hbm.at[src_v.at[j % _BLK]], rows_v.at[rb], sem_g).wait()

        pltpu.make_async_copy(w_hbm.at[sid, j], w_v.at[wb], sem_w).wait()
        pltpu.make_async_copy(dst_hbm.at[sid, j], dst_v.at[b], sem_d).wait()

        @pl.when(j >= 1)
        def _():
            pltpu.make_async_copy(
                rows_v.at[rbn], acc.at[dst_v.at[bn]], sem_s).wait()

        @pl.when((jn < _NCH) & (jn % _BLK == 0))
        def _():
            pltpu.sync_copy(src_hbm.at[sid, jn // _BLK], src_v)

        @pl.when((jn < _NCH) & (cid == 0))
        def _():
            pltpu.async_copy(z0_hbm.at[src_v.at[jn % _BLK]], rows_v.at[rbn],
                             sem_g)

        @pl.when((jn < _NCH) & (cid == 1))
        def _():
            pltpu.async_copy(z1_hbm.at[src_v.at[jn % _BLK]], rows_v.at[rbn],
                             sem_g)

        @pl.when(jn < _NCH)
        def _():
            pltpu.async_copy(w_hbm.at[sid, jn], w_v.at[wbn], sem_w)
            pltpu.async_copy(dst_hbm.at[sid, jn], dst_v.at[bn], sem_d)

        wrow = b * 16
        rrow = b * _CH
        for r in range(_CH):
            wbc = w_v[wrow + r // 8, pl.ds((r % 8) * 16, 16)]
            for g in range(_HH // 16):
                sl = pl.ds(g * 16, 16)
                rows_v[rrow + r, sl] = rows_v[rrow + r, sl] * wbc

        pltpu.async_copy(rows_v.at[rb], acc.at[dst_v.at[b]], sem_s, add=True)

        @pl.when(cid == 0)
        def _():
            pltpu.sync_copy(ones_v, cnt.at[dst_v.at[b]], add=True)
        return carry

    lax.fori_loop(0, _NCH, _chunk, 0)

    lastb = (_NCH - 1) % 2
    pltpu.make_async_copy(
        rows_v.at[pl.ds(lastb * _CH, _CH)], acc.at[dst_v.at[lastb]],
        sem_s).wait()
    plsc.subcore_barrier()

    rows = pl.ds(sid * _RPT, _RPT)

    @pl.when(cid == 0)
    def _():
        pltpu.sync_copy(acc.at[rows], sums0_out.at[rows])
        pltpu.sync_copy(cnt.at[rows], cnt_out.at[rows])

    @pl.when(cid == 1)
    def _():
        pltpu.sync_copy(acc.at[rows], sums1_out.at[rows])


def _sc_scatter(z0, z1, src_r, dst_r, w_r):
    mesh = plsc.VectorSubcoreMesh(core_axis_name="c", subcore_axis_name="s")
    run = pl.kernel(
        _sc_body,
        out_type=(
            jax.ShapeDtypeStruct((_NPAD, _HH), jnp.float32),
            jax.ShapeDtypeStruct((_NPAD, _HH), jnp.float32),
            jax.ShapeDtypeStruct((_NPAD,), jnp.float32),
        ),
        mesh=mesh,
        compiler_params=pltpu.CompilerParams(needs_layout_passes=False),
        scratch_types=[
            pltpu.VMEM((_BLK, _CH), jnp.int32),
            pltpu.VMEM((2, _CH), jnp.int32),
            pltpu.VMEM((32, 128), jnp.float32),
            pltpu.VMEM((2 * _CH, _HH), jnp.float32),
            pltpu.VMEM((_CH,), jnp.float32),
            pltpu.VMEM((_RPT,), jnp.float32),
            pltpu.VMEM_SHARED((_NPAD, _HH), jnp.float32),
            pltpu.VMEM_SHARED((_NPAD,), jnp.float32),
            pltpu.SemaphoreType.DMA,
            pltpu.SemaphoreType.DMA,
            pltpu.SemaphoreType.DMA,
            pltpu.SemaphoreType.DMA,
        ],
    )
    return run(z0, z1, src_r, dst_r, w_r)


def _xpart_body(x_ref, w2a_ref, p_ref):
    p_ref[:] = jnp.dot(x_ref[:], w2a_ref[:],
                       preferred_element_type=jnp.float32)


def _xpart(x, W2a):
    return pl.pallas_call(
        _xpart_body,
        out_shape=jax.ShapeDtypeStruct((_N, _H), jnp.float32),
    )(x, W2a)


def _update_body(p_ref, s0_ref, s1_ref, cnt_ref, w2b0_ref, w2b1_ref,
                 b2_ref, g2_ref, beta2_ref, mm2_ref, mv2_ref, out_ref):
    cnt = cnt_ref[:]
    inv = jnp.where(cnt > 0.0, 1.0 / jnp.maximum(cnt, 1.0), 0.0)
    a0 = s0_ref[:] * inv
    a1 = s1_ref[:] * inv
    y = (p_ref[:]
         + jnp.dot(a0, w2b0_ref[:], preferred_element_type=jnp.float32)
         + jnp.dot(a1, w2b1_ref[:], preferred_element_type=jnp.float32))
    s = g2_ref[:] * lax.rsqrt(mv2_ref[:] + _EPS)
    out = (y + b2_ref[:] - mm2_ref[:]) * s + beta2_ref[:]
    out_ref[:] = jnp.maximum(out, 0.0)


def _update(p, sums0, sums1, cnt, W2, b2, g2, beta2, mm2, mv2):
    return pl.pallas_call(
        _update_body,
        out_shape=jax.ShapeDtypeStruct((_N, _H), jnp.float32),
    )(p, sums0, sums1, cnt.reshape(_N, 1),
      W2[_D:_D + _HH], W2[_D + _HH:],
      b2.reshape(1, _H), g2.reshape(1, _H), beta2.reshape(1, _H),
      mm2.reshape(1, _H), mv2.reshape(1, _H))


def kernel(node_repesentations, edges, edge_weights,
           W1, b1, g1, beta1, mm1, mv1,
           W2, b2, g2, beta2, mm2, mv2):
    x = node_repesentations
    dst = edges[0].astype(jnp.int32).reshape(_NT, _NCH, _CH)
    src = edges[1].astype(jnp.int32).reshape(_NT, _NCH // _BLK, _BLK, _CH)
    w_r = jnp.broadcast_to(edge_weights.reshape(_NT, _NCH, _CH, 1),
                           (_NT, _NCH, _CH, 16)).reshape(_NT, _NCH, 1280)
    w_r = jnp.pad(w_r, ((0, 0), (0, 0), (0, 768))).reshape(
        _NT, _NCH, 16, 128)

    z0, z1 = _prepare(x, W1, b1, g1, beta1, mm1, mv1)
    p = _xpart(x, W2[:_D])
    sums0, sums1, cnt = _sc_scatter(z0, z1, src, dst, w_r)
    return _update(p, sums0[:_N], sums1[:_N], cnt[:_N],
                   W2, b2, g2, beta2, mm2, mv2)


# speedup vs baseline: 1.0174x; 1.0174x over previous
import functools

import jax
import jax.numpy as jnp
from jax import lax
from jax.experimental import pallas as pl
from jax.experimental.pallas import tpu as pltpu
from jax.experimental.pallas import tpu_sc as plsc

_EPS = 1e-3

_N = 10000
_D = 256
_E = 160000
_H = 256
_HH = _H // 2
_NT = 16
_EPT = _E // _NT
_CH = 80
_NCH = _EPT // _CH
_NPAD = 10240
_RPT = _NPAD // _NT
_BLK = 25


def _prepare_body(x_ref, w1_ref, b1_ref, g1_ref, beta1_ref, mm1_ref, mv1_ref,
                  w2a_ref, z0_ref, z1_ref, p_ref):
    s = g1_ref[:] * lax.rsqrt(mv1_ref[:] + _EPS)
    w = w1_ref[:] * s
    b = (b1_ref[:] - mm1_ref[:]) * s + beta1_ref[:]
    x = x_ref[:]
    y = jnp.dot(x, w, preferred_element_type=jnp.float32) + b
    z = jnp.maximum(y, 0.0)
    z0_ref[:] = z[:, :_HH]
    z1_ref[:] = z[:, _HH:]
    p_ref[:] = jnp.dot(x, w2a_ref[:], preferred_element_type=jnp.float32)


def _prepare(x, W1, b1, g1, beta1, mm1, mv1, W2a):
    return pl.pallas_call(
        _prepare_body,
        out_shape=(
            jax.ShapeDtypeStruct((_N, _HH), jnp.float32),
            jax.ShapeDtypeStruct((_N, _HH), jnp.float32),
            jax.ShapeDtypeStruct((_N, _H), jnp.float32),
        ),
    )(x, W1, b1.reshape(1, _H), g1.reshape(1, _H), beta1.reshape(1, _H),
      mm1.reshape(1, _H), mv1.reshape(1, _H), W2a)


def _sc_body(z0_hbm, z1_hbm, src_hbm, dst_hbm, w_hbm,
             sums0_out, sums1_out, cnt_out,
             src_v, dst_v, w_v, rows_v, ones_v, zeros_v, acc, cnt,
             sem_g, sem_s, sem_w, sem_d):
    cid = lax.axis_index("c")
    sid = lax.axis_index("s")

    zeros16 = jnp.zeros((16,), jnp.float32)
    ones16 = jnp.ones((16,), jnp.float32)
    for k in range(_CH // 16):
        ones_v[pl.ds(k * 16, 16)] = ones16
    for k in range(_RPT // 16):
        zeros_v[pl.ds(k * 16, 16)] = zeros16
    for r in range(_CH):
        for g in range(_HH // 16):
            rows_v[r, pl.ds(g * 16, 16)] = zeros16

    for b in range(_RPT // _CH):
        pltpu.sync_copy(rows_v.at[pl.ds(0, _CH)],
                        acc.at[pl.ds(sid * _RPT + b * _CH, _CH)])
    pltpu.sync_copy(zeros_v, cnt.at[pl.ds(sid * _RPT, _RPT)])

    pltpu.sync_copy(src_hbm.at[sid, 0], src_v)
    pltpu.async_copy(w_hbm.at[sid, 0], w_v.at[pl.ds(0, 16)], sem_w)
    pltpu.async_copy(dst_hbm.at[sid, 0], dst_v.at[0], sem_d)

    @pl.when(cid == 0)
    def _():
        pltpu.async_copy(z0_hbm.at[src_v.at[0]], rows_v.at[pl.ds(0, _CH)],
                         sem_g)

    @pl.when(cid == 1)
    def _():
        pltpu.async_copy(z1_hbm.at[src_v.at[0]], rows_v.at[pl.ds(0, _CH)],
                         sem_g)

    plsc.subcore_barrier()

    def _chunk(j, carry):
        b = j % 2
        bn = (j + 1) % 2
        jn = j + 1
        rb = pl.ds(pl.multiple_of(b * _CH, 8), _CH)
        rbn = pl.ds(pl.multiple_of(bn * _CH, 8), _CH)
        wb = pl.ds(pl.multiple_of(b * 16, 8), 16)
        wbn = pl.ds(pl.multiple_of(bn * 16, 8), 16)

        @pl.when(cid == 0)
        def _():
            pltpu.make_async_copy(
                z0_hbm.at[src_v.at[j % _BLK]], rows_v.at[rb], sem_g).wait()

        @pl.when(cid == 1)
        def _():
            pltpu.make_async_copy(
                z1_hbm.at[src_v.at[j % _BLK]], rows_v.at[rb], sem_g).wait()

        pltpu.make_async_copy(w_hbm.at[sid, j], w_v.at[wb], sem_w).wait()
        pltpu.make_async_copy(dst_hbm.at[sid, j], dst_v.at[b], sem_d).wait()

        @pl.when(j >= 1)
        def _():
            pltpu.make_async_copy(
                rows_v.at[rbn], acc.at[dst_v.at[bn]], sem_s).wait()

        @pl.when((jn < _NCH) & (jn % _BLK == 0))
        def _():
            pltpu.sync_copy(src_hbm.at[sid, jn // _BLK], src_v)

        @pl.when((jn < _NCH) & (cid == 0))
        def _():
            pltpu.async_copy(z0_hbm.at[src_v.at[jn % _BLK]], rows_v.at[rbn],
                             sem_g)

        @pl.when((jn < _NCH) & (cid == 1))
        def _():
            pltpu.async_copy(z1_hbm.at[src_v.at[jn % _BLK]], rows_v.at[rbn],
                             sem_g)

        @pl.when(jn < _NCH)
        def _():
            pltpu.async_copy(w_hbm.at[sid, jn], w_v.at[wbn], sem_w)
            pltpu.async_copy(dst_hbm.at[sid, jn], dst_v.at[bn], sem_d)

        wrow = b * 16
        rrow = b * _CH
        for r in range(_CH):
            wbc = w_v[wrow + r // 8, pl.ds((r % 8) * 16, 16)]
            for g in range(_HH // 16):
                sl = pl.ds(g * 16, 16)
                rows_v[rrow + r, sl] = rows_v[rrow + r, sl] * wbc

        pltpu.async_copy(rows_v.at[rb], acc.at[dst_v.at[b]], sem_s, add=True)

        @pl.when(cid == 0)
        def _():
            pltpu.sync_copy(ones_v, cnt.at[dst_v.at[b]], add=True)
        return carry

    lax.fori_loop(0, _NCH, _chunk, 0)

    lastb = (_NCH - 1) % 2
    pltpu.make_async_copy(
        rows_v.at[pl.ds(lastb * _CH, _CH)], acc.at[dst_v.at[lastb]],
        sem_s).wait()
    plsc.subcore_barrier()

    rows = pl.ds(sid * _RPT, _RPT)

    @pl.when(cid == 0)
    def _():
        pltpu.sync_copy(acc.at[rows], sums0_out.at[rows])
        pltpu.sync_copy(cnt.at[rows], cnt_out.at[rows])

    @pl.when(cid == 1)
    def _():
        pltpu.sync_copy(acc.at[rows], sums1_out.at[rows])


def _sc_scatter(z0, z1, src_r, dst_r, w_r):
    mesh = plsc.VectorSubcoreMesh(core_axis_name="c", subcore_axis_name="s")
    run = pl.kernel(
        _sc_body,
        out_type=(
            jax.ShapeDtypeStruct((_NPAD, _HH), jnp.float32),
            jax.ShapeDtypeStruct((_NPAD, _HH), jnp.float32),
            jax.ShapeDtypeStruct((_NPAD,), jnp.float32),
        ),
        mesh=mesh,
        compiler_params=pltpu.CompilerParams(needs_layout_passes=False),
        scratch_types=[
            pltpu.VMEM((_BLK, _CH), jnp.int32),
            pltpu.VMEM((2, _CH), jnp.int32),
            pltpu.VMEM((32, 128), jnp.float32),
            pltpu.VMEM((2 * _CH, _HH), jnp.float32),
            pltpu.VMEM((_CH,), jnp.float32),
            pltpu.VMEM((_RPT,), jnp.float32),
            pltpu.VMEM_SHARED((_NPAD, _HH), jnp.float32),
            pltpu.VMEM_SHARED((_NPAD,), jnp.float32),
            pltpu.SemaphoreType.DMA,
            pltpu.SemaphoreType.DMA,
            pltpu.SemaphoreType.DMA,
            pltpu.SemaphoreType.DMA,
        ],
    )
    return run(z0, z1, src_r, dst_r, w_r)


def _update_body(p_ref, s0_ref, s1_ref, cnt_ref, w2b0_ref, w2b1_ref,
                 b2_ref, g2_ref, beta2_ref, mm2_ref, mv2_ref, out_ref):
    cnt = cnt_ref[pl.ds(0, _N), :]
    inv = jnp.where(cnt > 0.0, 1.0 / jnp.maximum(cnt, 1.0), 0.0)
    a0 = s0_ref[pl.ds(0, _N), :] * inv
    a1 = s1_ref[pl.ds(0, _N), :] * inv
    y = (p_ref[:]
         + jnp.dot(a0, w2b0_ref[:], preferred_element_type=jnp.float32)
         + jnp.dot(a1, w2b1_ref[:], preferred_element_type=jnp.float32))
    s = g2_ref[:] * lax.rsqrt(mv2_ref[:] + _EPS)
    out = (y + b2_ref[:] - mm2_ref[:]) * s + beta2_ref[:]
    out_ref[:] = jnp.maximum(out, 0.0)


def _update(p, sums0, sums1, cnt, W2, b2, g2, beta2, mm2, mv2):
    return pl.pallas_call(
        _update_body,
        out_shape=jax.ShapeDtypeStruct((_N, _H), jnp.float32),
    )(p, sums0, sums1, cnt.reshape(_NPAD, 1),
      W2[_D:_D + _HH], W2[_D + _HH:],
      b2.reshape(1, _H), g2.reshape(1, _H), beta2.reshape(1, _H),
      mm2.reshape(1, _H), mv2.reshape(1, _H))


def kernel(node_repesentations, edges, edge_weights,
           W1, b1, g1, beta1, mm1, mv1,
           W2, b2, g2, beta2, mm2, mv2):
    x = node_repesentations
    dst = edges[0].astype(jnp.int32).reshape(_NT, _NCH, _CH)
    src = edges[1].astype(jnp.int32).reshape(_NT, _NCH // _BLK, _BLK, _CH)
    w_r = jnp.broadcast_to(edge_weights.reshape(_NT, _NCH, _CH, 1),
                           (_NT, _NCH, _CH, 16)).reshape(_NT, _NCH, 1280)
    w_r = jnp.pad(w_r, ((0, 0), (0, 0), (0, 768))).reshape(
        _NT, _NCH, 16, 128)

    z0, z1, p = _prepare(x, W1, b1, g1, beta1, mm1, mv1, W2[:_D])
    sums0, sums1, cnt = _sc_scatter(z0, z1, src, dst, w_r)
    return _update(p, sums0, sums1, cnt, W2, b2, g2, beta2, mm2, mv2)
